# Initial kernel scaffold; baseline (speedup 1.0000x reference)
import jax
import jax.numpy as jnp
from jax.experimental import pallas as pl


def kernel(x, edge_index, W, a):
    raise NotImplementedError("write your pallas kernel here")


# speedup vs baseline: 25.2720x; 25.2720x over previous
import functools

import jax
import jax.numpy as jnp
from jax import lax
from jax.experimental import pallas as pl
from jax.experimental.pallas import tpu as pltpu
from jax.experimental.pallas import tpu_sc as plsc

N = 10000
E = 160000
NFEAT = 256
NHID = 64
NHEADS = 4
ALPHA = 0.2

NC = 2
NT = 16
L = 16

HALF = 2 * NHID
DENW = 16

EPT = E // NT
CHUNK = 80
NCHUNK = EPT // CHUNK
RCH = 40
NRC = N // RCH
RPT = (NRC + NT - 1) // NT

BLKN = 2000

_SC_PARAMS = pltpu.CompilerParams(
    use_tc_tiling_on_sc=False, needs_layout_passes=False)


def _tc_body(x_ref, w2_ref, wa2_ref, wh_ref, s_ref):
    xb = x_ref[...]
    wh_ref[...] = jnp.dot(xb, w2_ref[0], preferred_element_type=jnp.float32)
    s_ref[...] = jnp.dot(xb, wa2_ref[0], preferred_element_type=jnp.float32)


_tc_call = pl.pallas_call(
    _tc_body,
    grid=(NC, N // BLKN),
    in_specs=[
        pl.BlockSpec((BLKN, NFEAT), lambda h, j: (j, 0)),
        pl.BlockSpec((1, NFEAT, HALF), lambda h, j: (h, 0, 0)),
        pl.BlockSpec((1, NFEAT, 4), lambda h, j: (h, 0, 0)),
    ],
    out_specs=[
        pl.BlockSpec((BLKN, HALF), lambda h, j: (h * (N // BLKN) + j, 0)),
        pl.BlockSpec((BLKN, 4), lambda h, j: (h * (N // BLKN) + j, 0)),
    ],
    out_shape=[
        jax.ShapeDtypeStruct((NC * N, HALF), jnp.float32),
        jax.ShapeDtypeStruct((NC * N, 4), jnp.float32),
    ],
)


_mesh = plsc.VectorSubcoreMesh(core_axis_name="c", subcore_axis_name="s")


@functools.partial(
    pl.kernel,
    out_type=(
        jax.ShapeDtypeStruct((NC * E,), jnp.float32),
        jax.ShapeDtypeStruct((NC * E,), jnp.float32),
    ),
    mesh=_mesh,
    compiler_params=_SC_PARAMS,
    scratch_types=[
        pltpu.VMEM((N, 4), jnp.float32),
        pltpu.VMEM((CHUNK,), jnp.int32),
        pltpu.VMEM((CHUNK,), jnp.int32),
        pltpu.VMEM((CHUNK,), jnp.float32),
        pltpu.VMEM((CHUNK,), jnp.float32),
    ],
)
def _sc_weights(src_hbm, dst_hbm, s_hbm, w0_hbm, w1_hbm,
                s_v, si_v, di_v, w0_v, w1_v):
    c = lax.axis_index("c")
    t = lax.axis_index("s")

    pltpu.sync_copy(s_hbm.at[pl.ds(c * N, N)], s_v)

    def _chunk(i, _):
        eoff = t * EPT + i * CHUNK
        pltpu.sync_copy(src_hbm.at[pl.ds(eoff, CHUNK)], si_v)
        pltpu.sync_copy(dst_hbm.at[pl.ds(eoff, CHUNK)], di_v)
        for k in range(CHUNK // L):
            s16 = si_v[pl.ds(k * L, L)]
            d16 = di_v[pl.ds(k * L, L)]
            col0 = jnp.zeros((L,), jnp.int32)
            z0 = (plsc.load_gather(s_v, [s16, col0])
                  + plsc.load_gather(s_v, [d16, col0 + 2]))
            w0_v[pl.ds(k * L, L)] = jnp.exp(jnp.maximum(z0, ALPHA * z0))
            z1 = (plsc.load_gather(s_v, [s16, col0 + 1])
                  + plsc.load_gather(s_v, [d16, col0 + 3]))
            w1_v[pl.ds(k * L, L)] = jnp.exp(jnp.maximum(z1, ALPHA * z1))
        pltpu.sync_copy(w0_v, w0_hbm.at[pl.ds(c * E + eoff, CHUNK)])
        pltpu.sync_copy(w1_v, w1_hbm.at[pl.ds(c * E + eoff, CHUNK)])
        return 0

    lax.fori_loop(0, NCHUNK, _chunk, 0)


@functools.partial(
    pl.kernel,
    out_type=jax.ShapeDtypeStruct((NC * N, HALF), jnp.float32),
    mesh=_mesh,
    compiler_params=_SC_PARAMS,
    scratch_types=[
        pltpu.VMEM((CHUNK,), jnp.int32),
        pltpu.VMEM((CHUNK,), jnp.int32),
        pltpu.VMEM((CHUNK,), jnp.int32),
        pltpu.VMEM((CHUNK,), jnp.float32),
        pltpu.VMEM((CHUNK,), jnp.float32),
        pltpu.VMEM((CHUNK, HALF), jnp.float32),
        pltpu.VMEM((CHUNK, DENW), jnp.float32),
        pltpu.VMEM((RCH, HALF), jnp.float32),
        pltpu.VMEM((RCH, DENW), jnp.float32),
        pltpu.VMEM_SHARED((N, HALF), jnp.float32),
        pltpu.VMEM_SHARED((N, DENW), jnp.float32),
        pltpu.SemaphoreType.DMA,
    ],
)
def _sc_scatter(src_hbm, dst_hbm, wh_hbm, w0_hbm, w1_hbm, out_hbm,
                si_v, di_v, gi_v, w0_v, w1_v, rows_v, den_v,
                nin_v, dnin_v, acc_num, acc_den, sem):
    c = lax.axis_index("c")
    t = lax.axis_index("s")
    lanes = lax.iota(jnp.int32, L)
    unit0 = (lanes == 0).astype(jnp.float32)
    unit1 = (lanes == 1).astype(jnp.float32)
    zeros16 = jnp.zeros((L,), jnp.float32)

    def _zrow(i, _):
        for k in range(HALF // L):
            nin_v[i, pl.ds(k * L, L)] = zeros16
        dnin_v[i, pl.ds(0, L)] = zeros16
        return 0
    lax.fori_loop(0, RCH, _zrow, 0)

    def _zcopy(q, _):
        g = q * NT + t
        @pl.when(g < NRC)
        def _():
            pltpu.sync_copy(nin_v, acc_num.at[pl.ds(g * RCH, RCH)])
            pltpu.sync_copy(dnin_v, acc_den.at[pl.ds(g * RCH, RCH)])
        return 0
    lax.fori_loop(0, RPT, _zcopy, 0)
    plsc.subcore_barrier()

    cbias = c * N

    def _chunk(i, _):
        eoff = t * EPT + i * CHUNK
        pltpu.sync_copy(src_hbm.at[pl.ds(eoff, CHUNK)], si_v)
        pltpu.sync_copy(dst_hbm.at[pl.ds(eoff, CHUNK)], di_v)
        pltpu.sync_copy(w0_hbm.at[pl.ds(c * E + eoff, CHUNK)], w0_v)
        pltpu.sync_copy(w1_hbm.at[pl.ds(c * E + eoff, CHUNK)], w1_v)
        for k in range(CHUNK // L):
            gi_v[pl.ds(k * L, L)] = si_v[pl.ds(k * L, L)] + cbias
        pltpu.async_copy(wh_hbm.at[gi_v], rows_v, sem).wait()

        def _scale(j, _):
            j16 = jnp.full((L,), j, jnp.int32)
            w0s = plsc.load_gather(w0_v, [j16])
            w1s = plsc.load_gather(w1_v, [j16])
            for k in range(NHID // L):
                rows_v[j, pl.ds(k * L, L)] = rows_v[j, pl.ds(k * L, L)] * w0s
                rows_v[j, pl.ds(NHID + k * L, L)] = (
                    rows_v[j, pl.ds(NHID + k * L, L)] * w1s)
            den_v[j, pl.ds(0, L)] = w0s * unit0 + w1s * unit1
            return 0
        lax.fori_loop(0, CHUNK, _scale, 0)

        pltpu.sync_copy(rows_v, acc_num.at[di_v], add=True)
        pltpu.sync_copy(den_v, acc_den.at[di_v], add=True)
        return 0

    lax.fori_loop(0, NCHUNK, _chunk, 0)
    plsc.subcore_barrier()

    def _nchunk(q, _):
        g = q * NT + t
        @pl.when(g < NRC)
        def _():
            roff = g * RCH
            pltpu.sync_copy(acc_num.at[pl.ds(roff, RCH)], nin_v)
            pltpu.sync_copy(acc_den.at[pl.ds(roff, RCH)], dnin_v)

            def _nrow(i, _):
                i16 = jnp.full((L,), i, jnp.int32)
                d0 = plsc.load_gather(dnin_v, [i16, jnp.zeros((L,), jnp.int32)])
                d1 = plsc.load_gather(dnin_v, [i16, jnp.ones((L,), jnp.int32)])
                r0 = 1.0 / jnp.maximum(d0, 1e-9)
                r1 = 1.0 / jnp.maximum(d1, 1e-9)
                for k in range(NHID // L):
                    nin_v[i, pl.ds(k * L, L)] = nin_v[i, pl.ds(k * L, L)] * r0
                    nin_v[i, pl.ds(NHID + k * L, L)] = (
                        nin_v[i, pl.ds(NHID + k * L, L)] * r1)
                return 0
            lax.fori_loop(0, RCH, _nrow, 0)
            pltpu.sync_copy(nin_v, out_hbm.at[pl.ds(c * N + roff, RCH)])
        return 0

    lax.fori_loop(0, RPT, _nchunk, 0)


def kernel(x, edge_index, W, a):
    src = edge_index[0]
    dst = edge_index[1]
    Wc = W.transpose(1, 0, 2).reshape(NFEAT, NHEADS * NHID)
    W2 = Wc.reshape(NFEAT, NC, HALF).transpose(1, 0, 2)
    u = jnp.einsum("hfk,hk->hf", W, a[:, :NHID])
    v = jnp.einsum("hfk,hk->hf", W, a[:, NHID:])
    wa = jnp.stack([
        jnp.stack([u[0], u[1], v[0], v[1]], axis=1),
        jnp.stack([u[2], u[3], v[2], v[3]], axis=1),
    ])

    wh2, s2 = _tc_call(x, W2, wa)
    w0, w1 = _sc_weights(src, dst, s2)
    out2 = _sc_scatter(src, dst, wh2, w0, w1)
    return jnp.concatenate([out2[:N], out2[N:]], axis=1)
